# initial kernel scaffold (unmeasured)
import jax
import jax.numpy as jnp
from jax import lax
from jax.experimental import pallas as pl
from jax.experimental.pallas import tpu as pltpu

N_DEV = 8


def kernel(x, w_mat, scale_x, scale_w):
    m_total, k_per = x.shape
    k_total, n = w_mat.shape
    m_per = m_total // N_DEV

    def body(x_ref, w_ref, sx_ref, sw_ref, out_ref, comm_ref,
             send_sems, recv_sems):
        my = lax.axis_index("i")

        barrier_sem = pltpu.get_barrier_semaphore()
        for p in range(1, N_DEV):
            pl.semaphore_signal(
                barrier_sem, inc=1,
                device_id=((my + p) % N_DEV,),
                device_id_type=pl.DeviceIdType.MESH,
            )
        pl.semaphore_wait(barrier_sem, N_DEV - 1)

        sends = []
        for h in range(1, N_DEV):
            tgt = (my + h) % N_DEV
            rdma = pltpu.make_async_remote_copy(
                src_ref=x_ref.at[pl.ds(tgt * m_per, m_per), :],
                dst_ref=comm_ref.at[h - 1],
                send_sem=send_sems.at[h - 1],
                recv_sem=recv_sems.at[h - 1],
                device_id=(tgt,),
                device_id_type=pl.DeviceIdType.MESH,
            )
            rdma.start()
            sends.append(rdma)

        scale = sx_ref[0] * sw_ref[0]

        out_ref[...] = jnp.dot(
            x_ref[pl.ds(my * m_per, m_per), :],
            w_ref[pl.ds(my * k_per, k_per), :],
            preferred_element_type=jnp.float32,
        )

        for h in range(1, N_DEV):
            src = (my - h) % N_DEV
            recv = pltpu.make_async_remote_copy(
                src_ref=comm_ref.at[h - 1],
                dst_ref=comm_ref.at[h - 1],
                send_sem=send_sems.at[h - 1],
                recv_sem=recv_sems.at[h - 1],
                device_id=(src,),
                device_id_type=pl.DeviceIdType.MESH,
            )
            recv.wait_recv()
            out_ref[...] += jnp.dot(
                comm_ref[h - 1],
                w_ref[pl.ds(src * k_per, k_per), :],
                preferred_element_type=jnp.float32,
            )

        out_ref[...] *= scale

        for rdma in sends:
            rdma.wait_send()

    return pl.pallas_call(
        body,
        out_shape=jax.ShapeDtypeStruct((m_per, n), jnp.float32),
        in_specs=[
            pl.BlockSpec(memory_space=pltpu.VMEM),
            pl.BlockSpec(memory_space=pltpu.VMEM),
            pl.BlockSpec(memory_space=pltpu.SMEM),
            pl.BlockSpec(memory_space=pltpu.SMEM),
        ],
        out_specs=pl.BlockSpec(memory_space=pltpu.VMEM),
        scratch_shapes=[
            pltpu.VMEM((N_DEV - 1, m_per, k_per), x.dtype),
            pltpu.SemaphoreType.DMA((N_DEV - 1,)),
            pltpu.SemaphoreType.DMA((N_DEV - 1,)),
        ],
        compiler_params=pltpu.CompilerParams(collective_id=0),
    )(x, w_mat, scale_x, scale_w)


# baseline (device time: 68596 ns/iter reference)
import jax
import jax.numpy as jnp
from jax import lax
from jax.experimental import pallas as pl
from jax.experimental.pallas import tpu as pltpu

N_DEV = 8
SUB = 2


def kernel(x, w_mat, scale_x, scale_w):
    m_total, k_per = x.shape
    k_total, n = w_mat.shape
    m_per = m_total // N_DEV
    kb = k_per // SUB
    n_steps = N_DEV * SUB
    NBUF = 3

    def body(x_ref, w_hbm, sx_ref, sw_ref, out_ref, x8_ref, comm_ref,
             w_bufs, w_dma_sems, send_sems, recv_sems):
        my = lax.axis_index("i")

        def w_dma(t):
            h, s = t // SUB, t % SUB
            src = (my - h) % N_DEV
            return pltpu.make_async_copy(
                w_hbm.at[pl.ds(src * k_per + s * kb, kb), :],
                w_bufs.at[t % NBUF],
                w_dma_sems.at[t % NBUF],
            )

        for t in range(NBUF):
            w_dma(t).start()

        x8_ref[...] = x_ref[...].astype(jnp.float8_e4m3fn)

        barrier_sem = pltpu.get_barrier_semaphore()
        for p in range(1, N_DEV):
            pl.semaphore_signal(
                barrier_sem, inc=1,
                device_id=((my + p) % N_DEV,),
                device_id_type=pl.DeviceIdType.MESH,
            )
        pl.semaphore_wait(barrier_sem, N_DEV - 1)

        sends = []
        for h in range(1, N_DEV):
            tgt = (my + h) % N_DEV
            rdma = pltpu.make_async_remote_copy(
                src_ref=x8_ref.at[pl.ds(tgt * m_per, m_per), :],
                dst_ref=comm_ref.at[h - 1],
                send_sem=send_sems.at[h - 1],
                recv_sem=recv_sems.at[h - 1],
                device_id=(tgt,),
                device_id_type=pl.DeviceIdType.MESH,
            )
            rdma.start()
            sends.append(rdma)

        for t in range(n_steps):
            h, s = t // SUB, t % SUB
            w_dma(t).wait()
            if h == 0:
                a = x_ref[pl.ds(my * m_per, m_per), pl.ds(s * kb, kb)]
            else:
                if s == 0:
                    recv = pltpu.make_async_remote_copy(
                        src_ref=comm_ref.at[h - 1],
                        dst_ref=comm_ref.at[h - 1],
                        send_sem=send_sems.at[h - 1],
                        recv_sem=recv_sems.at[h - 1],
                        device_id=((my - h) % N_DEV,),
                        device_id_type=pl.DeviceIdType.MESH,
                    )
                    recv.wait_recv()
                a = comm_ref[h - 1][:, s * kb:(s + 1) * kb].astype(jnp.float32)
            contrib = jnp.dot(a, w_bufs[t % NBUF],
                              preferred_element_type=jnp.float32)
            if t == 0:
                out_ref[...] = contrib
            else:
                out_ref[...] += contrib
            if t + NBUF < n_steps:
                w_dma(t + NBUF).start()

        out_ref[...] *= sx_ref[0] * sw_ref[0]

        for rdma in sends:
            rdma.wait_send()

    return pl.pallas_call(
        body,
        out_shape=jax.ShapeDtypeStruct((m_per, n), jnp.float32),
        in_specs=[
            pl.BlockSpec(memory_space=pltpu.VMEM),
            pl.BlockSpec(memory_space=pl.ANY),
            pl.BlockSpec(memory_space=pltpu.SMEM),
            pl.BlockSpec(memory_space=pltpu.SMEM),
        ],
        out_specs=pl.BlockSpec(memory_space=pltpu.VMEM),
        scratch_shapes=[
            pltpu.VMEM((m_total, k_per), jnp.float8_e4m3fn),
            pltpu.VMEM((N_DEV - 1, m_per, k_per), jnp.float8_e4m3fn),
            pltpu.VMEM((NBUF, kb, n), jnp.float32),
            pltpu.SemaphoreType.DMA((NBUF,)),
            pltpu.SemaphoreType.DMA((N_DEV - 1,)),
            pltpu.SemaphoreType.DMA((N_DEV - 1,)),
        ],
        compiler_params=pltpu.CompilerParams(
            collective_id=0,
            vmem_limit_bytes=100 * 1024 * 1024,
        ),
    )(x, w_mat, scale_x, scale_w)
